# baseline (speedup 1.0000x reference)
import functools

import jax
import jax.numpy as jnp
from jax import lax
from jax.experimental import pallas as pl
from jax.experimental.pallas import tpu as pltpu
from jax.experimental.pallas import tpu_sc as plsc

D_MODEL = 64
VOCAB = 1000000
SCALE = 8.0

B_TOTAL = 4096 * 200
NC, NS = 2, 16
NW = NC * NS
B_PER_W = B_TOTAL // NW
CHUNK = 128
N_CHUNKS = B_PER_W // CHUNK
ROW_UNROLL = 8
LANES = 16


def _emb_kernel(x_hbm, table_hbm, out_hbm, idx_v, rows_v, gsem):
    wid = lax.axis_index("s") * NC + lax.axis_index("c")
    base = wid * B_PER_W
    pltpu.sync_copy(x_hbm.at[pl.ds(base, B_PER_W)], idx_v)

    def chunk_body(c, carry):
        off = c * CHUNK
        pltpu.async_copy(
            table_hbm.at[idx_v.at[pl.ds(off, CHUNK)]], rows_v, gsem
        ).wait()

        def scale_body(k, carry2):
            i0 = k * ROW_UNROLL
            for r in range(ROW_UNROLL):
                for j in range(D_MODEL // LANES):
                    sl = pl.ds(j * LANES, LANES)
                    rows_v[i0 + r, sl] = rows_v[i0 + r, sl] * SCALE
            return carry2

        lax.fori_loop(0, CHUNK // ROW_UNROLL, scale_body, 0, unroll=False)
        pltpu.sync_copy(rows_v, out_hbm.at[pl.ds(base + off, CHUNK)])
        return carry

    lax.fori_loop(0, N_CHUNKS, chunk_body, 0, unroll=False)


@jax.jit
def _emb(x_flat, table):
    mesh = plsc.VectorSubcoreMesh(core_axis_name="c", subcore_axis_name="s")
    f = functools.partial(
        pl.kernel,
        mesh=mesh,
        out_type=jax.ShapeDtypeStruct((B_TOTAL, D_MODEL), jnp.float32),
        scratch_types=[
            pltpu.VMEM((B_PER_W,), jnp.int32),
            pltpu.VMEM((CHUNK, D_MODEL), jnp.float32),
            pltpu.SemaphoreType.DMA,
        ],
        compiler_params=pltpu.CompilerParams(use_tc_tiling_on_sc=False),
    )(_emb_kernel)
    return f(x_flat, table)


def kernel(x, emb_weight):
    x_flat = x.reshape(-1).astype(jnp.int32)
    out = _emb(x_flat, emb_weight)
    return out.reshape(x.shape + (D_MODEL,))


# speedup vs baseline: 1.1585x; 1.1585x over previous
import functools

import jax
import jax.numpy as jnp
from jax import lax
from jax.experimental import pallas as pl
from jax.experimental.pallas import tpu as pltpu
from jax.experimental.pallas import tpu_sc as plsc

D_MODEL = 64
VOCAB = 1000000
SCALE = 8.0

B_TOTAL = 4096 * 200
NC, NS = 2, 16
NW = NC * NS
B_PER_W = B_TOTAL // NW
CHUNK = 128
N_CHUNKS = B_PER_W // CHUNK
NBUF = 4
PREF = 2
N_GROUPS = N_CHUNKS // NBUF
ROW_UNROLL = 8
LANES = 16


def _emb_kernel(x_hbm, table_hbm, out_hbm, idx_v, rows_v, gsem, osem):
    wid = lax.axis_index("s") * NC + lax.axis_index("c")
    base = wid * B_PER_W
    pltpu.sync_copy(x_hbm.at[pl.ds(base, B_PER_W)], idx_v)

    def start_gather(c, b):
        pltpu.async_copy(
            table_hbm.at[idx_v.at[pl.ds(c * CHUNK, CHUNK)]],
            rows_v.at[b],
            gsem.at[b],
        )

    def wait_gather(c, b):
        pltpu.make_async_copy(
            table_hbm.at[idx_v.at[pl.ds(c * CHUNK, CHUNK)]],
            rows_v.at[b],
            gsem.at[b],
        ).wait()

    def start_out(c, b):
        pltpu.async_copy(
            rows_v.at[b], out_hbm.at[pl.ds(base + c * CHUNK, CHUNK)], osem.at[b]
        )

    def wait_out(c, b):
        pltpu.make_async_copy(
            rows_v.at[b], out_hbm.at[pl.ds(base + c * CHUNK, CHUNK)], osem.at[b]
        ).wait()

    def scale(b):
        def scale_body(k, carry):
            i0 = k * ROW_UNROLL
            for r in range(ROW_UNROLL):
                for j in range(D_MODEL // LANES):
                    sl = pl.ds(j * LANES, LANES)
                    rows_v[b, i0 + r, sl] = rows_v[b, i0 + r, sl] * SCALE
            return carry

        lax.fori_loop(0, CHUNK // ROW_UNROLL, scale_body, 0, unroll=False)

    def step(c, b, wait_o, issue):
        b2 = (b + PREF) % NBUF
        if issue:
            if wait_o:
                wait_out(c + PREF - NBUF, b2)
            start_gather(c + PREF, b2)
        wait_gather(c, b)
        scale(b)
        start_out(c, b)

    for b in range(PREF):
        start_gather(b, b)
    for b in range(NBUF):
        step(b, b, wait_o=(b + PREF >= NBUF), issue=True)

    def group_body(g, carry):
        c0 = g * NBUF
        for b in range(NBUF):
            step(c0 + b, b, wait_o=True, issue=True)
        return carry

    lax.fori_loop(1, N_GROUPS - 1, group_body, 0, unroll=False)

    c0 = (N_GROUPS - 1) * NBUF
    for b in range(NBUF):
        step(c0 + b, b, wait_o=True, issue=(b < NBUF - PREF))
    for b in range(NBUF):
        wait_out(c0 + b, b)


@jax.jit
def _emb(x_flat, table):
    mesh = plsc.VectorSubcoreMesh(core_axis_name="c", subcore_axis_name="s")
    f = functools.partial(
        pl.kernel,
        mesh=mesh,
        out_type=jax.ShapeDtypeStruct((B_TOTAL, D_MODEL), jnp.float32),
        scratch_types=[
            pltpu.VMEM((B_PER_W,), jnp.int32),
            pltpu.VMEM((NBUF, CHUNK, D_MODEL), jnp.float32),
            pltpu.SemaphoreType.DMA((NBUF,)),
            pltpu.SemaphoreType.DMA((NBUF,)),
        ],
        compiler_params=pltpu.CompilerParams(use_tc_tiling_on_sc=False),
    )(_emb_kernel)
    return f(x_flat, table)


def kernel(x, emb_weight):
    x_flat = x.reshape(-1).astype(jnp.int32)
    out = _emb(x_flat, emb_weight)
    return out.reshape(x.shape + (D_MODEL,))
